# Initial kernel scaffold; baseline (speedup 1.0000x reference)
import functools

import jax
import jax.numpy as jnp
from jax import lax
from jax.experimental import pallas as pl
from jax.experimental.pallas import tpu as pltpu
from jax.experimental.pallas import tpu_sc as plsc

HIDDEN = 128
H_DIR = 64
SAMPLE = 16
N_LAYERS = 3
N_NODES = 16384
BATCH = 16
SEQ = 1024
EMB = 128

NC = 2
NS = 16
NW = NC * NS
ROWS_PER_W = N_NODES // NW

_SC_MESH = plsc.VectorSubcoreMesh(
    core_axis_name="c", subcore_axis_name="s", num_cores=NC, num_subcores=NS)

S_PAD = 16400


def _emb_gather_body(table_hbm, idx_hbm, out_hbm, idx_v, out_v, sem):
  wid = lax.axis_index("s") * NC + lax.axis_index("c")
  base = wid * ROWS_PER_W
  pltpu.sync_copy(idx_hbm.at[pl.ds(wid * 4, 4)], idx_v)
  cps = []
  for j in range(4):
    cps.append(pltpu.async_copy(
        table_hbm.at[idx_v.at[j]], out_v.at[pl.ds(j * 128, 128)], sem))
  for cp in cps:
    cp.wait()
  pltpu.sync_copy(out_v, out_hbm.at[pl.ds(base, ROWS_PER_W)])


def _emb_gather(table, idx2d):
  f = pl.kernel(
      _emb_gather_body,
      out_type=jax.ShapeDtypeStruct((N_NODES, EMB), jnp.float32),
      mesh=_SC_MESH,
      scratch_types=[
          pltpu.VMEM((4, 128), jnp.int32),
          pltpu.VMEM((ROWS_PER_W, EMB), jnp.float32),
          pltpu.SemaphoreType.DMA,
      ],
  )
  return f(table, idx2d)


_N_CHUNKS = 128
_CHUNK_ROWS = 64
_NODES_PER_CHUNK = 4


def _reduce_chunk(rows_v, b, j, out_v, idx_v, s_v, len_v, with_len):
  def node_body(k, _):
    node = j * _NODES_PER_CHUNK + k
    rbase = k * SAMPLE
    for cg in range(EMB // 16):
      acc = rows_v[b, rbase, pl.ds(cg * 16, 16)]
      for r in range(1, SAMPLE):
        acc = acc + rows_v[b, rbase + r, pl.ds(cg * 16, 16)]
      out_v[node, pl.ds(cg * 16, 16)] = acc
    if with_len:
      iv = idx_v[j, pl.ds(k * SAMPLE, SAMPLE)]
      sv = plsc.load_gather(s_v, [iv])
      lensum = jnp.sum(sv)
      len_v[node, :] = jnp.broadcast_to(lensum, (16,))
    return 0

  lax.fori_loop(0, _NODES_PER_CHUNK, node_body, 0)


def _neigh_sum_body(with_len, *refs):
  if with_len:
    (table_hbm, idx_hbm, s_hbm, sums_hbm, len_hbm,
     idx_v, rows_v, out_v, len_v, s_v, sems) = refs
  else:
    (table_hbm, idx_hbm, sums_hbm,
     idx_v, rows_v, out_v, sems) = refs
    s_v = len_v = s_hbm = len_hbm = None

  wid = lax.axis_index("s") * NC + lax.axis_index("c")
  base = wid * ROWS_PER_W
  pltpu.sync_copy(idx_hbm.at[pl.ds(wid * _N_CHUNKS, _N_CHUNKS)], idx_v)
  if with_len:
    pltpu.sync_copy(s_hbm, s_v)

  def start(j, b):
    return pltpu.async_copy(table_hbm.at[idx_v.at[j]], rows_v.at[b],
                            sems.at[b])

  def wait(b):
    pltpu.make_async_copy(rows_v.at[b], rows_v.at[b], sems.at[b]).wait()

  start(0, 0)

  def outer(g, _):
    for bb in range(2):
      j = g * 2 + bb
      nxt = j + 1

      @pl.when(nxt < _N_CHUNKS)
      def _():
        start(nxt, 1 - bb)

      wait(bb)
      _reduce_chunk(rows_v, bb, j, out_v, idx_v, s_v, len_v, with_len)
    return 0

  lax.fori_loop(0, _N_CHUNKS // 2, outer, 0)
  pltpu.sync_copy(out_v, sums_hbm.at[pl.ds(base, ROWS_PER_W)])
  if with_len:
    pltpu.sync_copy(len_v, len_hbm.at[pl.ds(base, ROWS_PER_W)])


def _neigh_sum(table, idx_staged, s_full=None):
  with_len = s_full is not None
  out_type = [jax.ShapeDtypeStruct((N_NODES, EMB), jnp.float32)]
  scratch = [
      pltpu.VMEM((_N_CHUNKS, _CHUNK_ROWS), jnp.int32),
      pltpu.VMEM((2, _CHUNK_ROWS, EMB), jnp.float32),
      pltpu.VMEM((ROWS_PER_W, EMB), jnp.float32),
  ]
  args = [table, idx_staged]
  if with_len:
    out_type.append(jax.ShapeDtypeStruct((N_NODES, 16), jnp.float32))
    scratch.append(pltpu.VMEM((ROWS_PER_W, 16), jnp.float32))
    scratch.append(pltpu.VMEM((S_PAD,), jnp.float32))
    args.append(s_full)
  scratch.append(pltpu.SemaphoreType.DMA((2,)))
  f = pl.kernel(
      functools.partial(_neigh_sum_body, with_len),
      out_type=tuple(out_type) if with_len else out_type[0],
      mesh=_SC_MESH,
      scratch_types=scratch,
  )
  return f(*args)


_TBLK = 64
_NGRID = SEQ // _TBLK


def _lstm_body(xsf_ref, xsb_ref, wfT_ref, wbT_ref, bf_ref, bb_ref, wblk_ref,
               ysf_ref, ysb_ref, hf, cf, hb, cb, gf_s, gb_s):
  i = pl.program_id(0)

  @pl.when(i == 0)
  def _():
    hf[...] = jnp.zeros((BATCH, H_DIR), jnp.float32)
    cf[...] = jnp.zeros((BATCH, H_DIR), jnp.float32)
    hb[...] = jnp.zeros((BATCH, H_DIR), jnp.float32)
    cb[...] = jnp.zeros((BATCH, H_DIR), jnp.float32)

  xf = xsf_ref[...].reshape(_TBLK * BATCH, EMB)
  gf_s[...] = (jnp.dot(xf, wfT_ref[...], preferred_element_type=jnp.float32)
               + bf_ref[0:1, :]).reshape(_TBLK, BATCH, 4 * H_DIR)
  xb = xsb_ref[...].reshape(_TBLK * BATCH, EMB)
  gb_s[...] = (jnp.dot(xb, wbT_ref[...], preferred_element_type=jnp.float32)
               + bb_ref[0:1, :]).reshape(_TBLK, BATCH, 4 * H_DIR)

  def step(k, _):
    tb = _TBLK - 1 - k
    hcat = jnp.concatenate([hf[...], hb[...]], axis=1)
    g2 = jnp.dot(hcat, wblk_ref[...], preferred_element_type=jnp.float32)
    gfk = gf_s[k] + g2[:, :4 * H_DIR]
    gbk = gb_s[tb] + g2[:, 4 * H_DIR:]
    for g, h_r, c_r, ys_r, t in ((gfk, hf, cf, ysf_ref, k),
                                 (gbk, hb, cb, ysb_ref, tb)):
      ig = jax.nn.sigmoid(g[:, :H_DIR])
      fg = jax.nn.sigmoid(g[:, H_DIR:2 * H_DIR])
      gg = jnp.tanh(g[:, 2 * H_DIR:3 * H_DIR])
      og = jax.nn.sigmoid(g[:, 3 * H_DIR:])
      c2 = fg * c_r[...] + ig * gg
      h2 = og * jnp.tanh(c2)
      c_r[...] = c2
      h_r[...] = h2
      ys_r[t] = h2.reshape(1, BATCH, H_DIR)
    return 0

  lax.fori_loop(0, _TBLK, step, 0)


def _lstm_layer(xs, W_ih, W_hh, b_ih, b_hh, layer):
  wfT = W_ih[layer, 0].T
  wbT = W_ih[layer, 1].T
  bf = jnp.tile((b_ih[layer, 0] + b_hh[layer, 0])[None, :], (8, 1))
  bb = jnp.tile((b_ih[layer, 1] + b_hh[layer, 1])[None, :], (8, 1))
  wblk = jnp.zeros((2 * H_DIR, 8 * H_DIR), jnp.float32)
  wblk = wblk.at[:H_DIR, :4 * H_DIR].set(W_hh[layer, 0].T)
  wblk = wblk.at[H_DIR:, 4 * H_DIR:].set(W_hh[layer, 1].T)

  grid = (_NGRID,)
  blk = pl.BlockSpec((_TBLK, BATCH, EMB), lambda i: (i, 0, 0))
  blk_rev = pl.BlockSpec((_TBLK, BATCH, EMB), lambda i: (_NGRID - 1 - i, 0, 0))
  full = lambda shape: pl.BlockSpec(shape, lambda i: tuple(0 for _ in shape))
  oblk = pl.BlockSpec((_TBLK, BATCH, H_DIR), lambda i: (i, 0, 0))
  oblk_rev = pl.BlockSpec((_TBLK, BATCH, H_DIR),
                          lambda i: (_NGRID - 1 - i, 0, 0))
  return pl.pallas_call(
      _lstm_body,
      grid=grid,
      in_specs=[blk, blk_rev, full((EMB, 4 * H_DIR)), full((EMB, 4 * H_DIR)),
                full((8, 4 * H_DIR)), full((8, 4 * H_DIR)),
                full((2 * H_DIR, 8 * H_DIR))],
      out_specs=[oblk, oblk_rev],
      out_shape=[jax.ShapeDtypeStruct((SEQ, BATCH, H_DIR), jnp.float32),
                 jax.ShapeDtypeStruct((SEQ, BATCH, H_DIR), jnp.float32)],
      scratch_shapes=[pltpu.VMEM((BATCH, H_DIR), jnp.float32)] * 4
      + [pltpu.VMEM((_TBLK, BATCH, 4 * H_DIR), jnp.float32)] * 2,
      compiler_params=pltpu.CompilerParams(
          dimension_semantics=("arbitrary",)),
  )(xs, xs, wfT, wbT, bf, bb, wblk)


def _row_sign_body(x_ref, ones_ref, out_ref):
  r = jnp.dot(jax.nn.relu(x_ref[...]), ones_ref[...],
              preferred_element_type=jnp.float32)
  out_ref[...] = (r > 0.0).astype(jnp.float32)


def _row_sign(x):
  ones = jnp.ones((EMB, 8), jnp.float32)
  return pl.pallas_call(
      _row_sign_body,
      grid=(32,),
      in_specs=[pl.BlockSpec((512, EMB), lambda i: (i, 0)),
                pl.BlockSpec((EMB, 8), lambda i: (0, 0))],
      out_specs=pl.BlockSpec((512, 8), lambda i: (i, 0)),
      out_shape=jax.ShapeDtypeStruct((N_NODES, 8), jnp.float32),
  )(x, ones)


def _agg_body(h_ref, sums_ref, len_ref, w_ref, b_ref, out_ref):
  recip = 1.0 / jnp.maximum(len_ref[:, 0:1], 1.0)
  means = sums_ref[...] * recip
  g = (jnp.dot(h_ref[...], w_ref[:HIDDEN, :],
               preferred_element_type=jnp.float32)
       + jnp.dot(means, w_ref[HIDDEN:, :], preferred_element_type=jnp.float32)
       + b_ref[0:1, :])
  out_ref[...] = jnp.maximum(g, 0.0)


def _agg(h, sums, len16, W, b):
  b8 = jnp.tile(b[None, :], (8, 1))
  return pl.pallas_call(
      _agg_body,
      grid=(32,),
      in_specs=[pl.BlockSpec((512, HIDDEN), lambda i: (i, 0)),
                pl.BlockSpec((512, HIDDEN), lambda i: (i, 0)),
                pl.BlockSpec((512, 16), lambda i: (i, 0)),
                pl.BlockSpec((2 * HIDDEN, HIDDEN), lambda i: (0, 0)),
                pl.BlockSpec((8, HIDDEN), lambda i: (0, 0))],
      out_specs=pl.BlockSpec((512, HIDDEN), lambda i: (i, 0)),
      out_shape=jax.ShapeDtypeStruct((N_NODES, HIDDEN), jnp.float32),
  )(h, sums, len16, W, b8)


def kernel(fw_adj_info, bw_adj_info, feature_info, batch_nodes, embedding,
           lstm_W_ih, lstm_W_hh, lstm_b_ih, lstm_b_hh, padding_vector,
           fw_agg_W, fw_agg_b, bw_agg_W, bw_agg_b):
  bsz, seq = batch_nodes.shape

  feat2d = feature_info[:-1].reshape(-1).astype(jnp.int32).reshape(128, 128)
  x = _emb_gather(embedding, feat2d)

  xs = x.reshape(bsz, seq, EMB).transpose(1, 0, 2)
  for layer in range(2):
    ysf, ysb = _lstm_layer(xs, lstm_W_ih, lstm_W_hh, lstm_b_ih, lstm_b_hh,
                           layer)
    xs = jnp.concatenate([ysf, ysb], axis=-1)
  feature_vector = xs.transpose(1, 0, 2).reshape(N_NODES, HIDDEN)

  node_repres = jnp.concatenate([feature_vector, padding_vector], axis=0)

  s8 = _row_sign(feature_vector)
  s_pad = (jnp.sum(jax.nn.relu(padding_vector)) > 0.0).astype(jnp.float32)
  s_full = jnp.zeros((S_PAD,), jnp.float32)
  s_full = s_full.at[:N_NODES].set(s8[:, 0]).at[N_NODES].set(s_pad)

  fw_idx = fw_adj_info.astype(jnp.int32).reshape(NW * _N_CHUNKS, _CHUNK_ROWS)
  bw_idx = bw_adj_info.astype(jnp.int32).reshape(NW * _N_CHUNKS, _CHUNK_ROWS)

  sums_f, len_f = _neigh_sum(node_repres, fw_idx, s_full)
  sums_b, len_b = _neigh_sum(node_repres, bw_idx, s_full)

  fw_h = _agg(feature_vector, sums_f, len_f, fw_agg_W[0], fw_agg_b[0])
  bw_h = _agg(feature_vector, sums_b, len_b, bw_agg_W[0], bw_agg_b[0])

  zero_row = jnp.zeros((1, HIDDEN), jnp.float32)
  for layer in range(1, N_LAYERS):
    tf = jnp.concatenate([fw_h, zero_row], axis=0)
    sums_f = _neigh_sum(tf, fw_idx)
    fw_h = _agg(fw_h, sums_f, len_f, fw_agg_W[layer], fw_agg_b[layer])
    tb = jnp.concatenate([bw_h, zero_row], axis=0)
    sums_b = _neigh_sum(tb, bw_idx)
    bw_h = _agg(bw_h, sums_b, len_b, bw_agg_W[layer], bw_agg_b[layer])

  return jnp.concatenate([fw_h.reshape(bsz, seq, HIDDEN),
                          bw_h.reshape(bsz, seq, HIDDEN)], axis=-1)


# speedup vs baseline: 6.0063x; 6.0063x over previous
import functools

import jax
import jax.numpy as jnp
from jax import lax
from jax.experimental import pallas as pl
from jax.experimental.pallas import tpu as pltpu
from jax.experimental.pallas import tpu_sc as plsc

HIDDEN = 128
H_DIR = 64
SAMPLE = 16
N_LAYERS = 3
N_NODES = 16384
BATCH = 16
SEQ = 1024
EMB = 128

NC = 2
NS = 16
NW = NC * NS
ROWS_PER_W = N_NODES // NW

_sc_mesh_cache = []


def _sc_mesh():
  if not _sc_mesh_cache:
    _sc_mesh_cache.append(plsc.VectorSubcoreMesh(
        core_axis_name="c", subcore_axis_name="s",
        num_cores=NC, num_subcores=NS))
  return _sc_mesh_cache[0]


_EMB_CHUNKS = 8


def _emb_gather_body(table_hbm, idx_hbm, out_hbm, idx_v, rows_v, sg, sw):
  wid = lax.axis_index("s") * NC + lax.axis_index("c")
  base = wid * ROWS_PER_W

  pltpu.sync_copy(idx_hbm.at[pl.ds(wid * 4, 4)], idx_v)

  def gstart(j, b):
    isl = idx_v.at[j // 2, pl.ds((j % 2) * 64, 64)]
    pltpu.async_copy(table_hbm.at[isl], rows_v.at[b], sg.at[b])

  def gwait(b):
    pltpu.make_async_copy(
        table_hbm.at[pl.ds(0, 64)], rows_v.at[b], sg.at[b]).wait()

  def wstart(j, b):
    pltpu.async_copy(rows_v.at[b], out_hbm.at[pl.ds(base + j * 64, 64)],
                     sw.at[b])

  def wwait(b):
    pltpu.make_async_copy(rows_v.at[b], out_hbm.at[pl.ds(0, 64)],
                          sw.at[b]).wait()

  gstart(0, 0)
  gstart(1, 1)

  def body(j, _):
    b = j % 2
    gwait(b)
    wstart(j, b)

    @pl.when(j + 2 < _EMB_CHUNKS)
    def _():
      wwait(b)
      gstart(j + 2, b)

    return 0

  lax.fori_loop(0, _EMB_CHUNKS, body, 0)
  wwait(0)
  wwait(1)


def _emb_gather(table, idx2d):
  f = pl.kernel(
      _emb_gather_body,
      out_type=jax.ShapeDtypeStruct((N_NODES, EMB), jnp.float32),
      mesh=_sc_mesh(),
      scratch_types=[
          pltpu.VMEM((4, 128), jnp.int32),
          pltpu.VMEM((2, 64, EMB), jnp.float32),
          pltpu.SemaphoreType.DMA((2,)),
          pltpu.SemaphoreType.DMA((2,)),
      ],
  )
  return f(table, idx2d)


_N_CHUNKS = 256
_CHUNK_ROWS = 32
_NODES_PER_CHUNK = 2


def _reduce_chunk(rows_v, srows_v, b, j, out_v, len_v, with_len):
  def node_body(k, _):
    rbase = k * SAMPLE
    for cg in range(EMB // 16):
      acc = rows_v[b, rbase, pl.ds(cg * 16, 16)]
      for r in range(1, SAMPLE):
        acc = acc + rows_v[b, rbase + r, pl.ds(cg * 16, 16)]
      out_v[b, k, pl.ds(cg * 16, 16)] = acc
    if with_len:
      accl = srows_v[b, rbase, pl.ds(0, 16)]
      for r in range(1, SAMPLE):
        accl = accl + srows_v[b, rbase + r, pl.ds(0, 16)]
      len_v[j * _NODES_PER_CHUNK + k, :] = accl
    return 0

  lax.fori_loop(0, _NODES_PER_CHUNK, node_body, 0)


def _neigh_phase(tbl, idxh, s_hbm, sumsh, lenh, wid, base,
                 idx_v, rows_v, srows_v, out_v, len_v, sg, sw, with_len):
  pltpu.sync_copy(idxh.at[pl.ds(wid * 64, 64)], idx_v)

  def gstart(j, b):
    isl = idx_v.at[j // 4, pl.ds((j % 4) * 32, 32)]
    pltpu.async_copy(tbl.at[isl], rows_v.at[b], sg.at[b])
    if with_len:
      pltpu.async_copy(s_hbm.at[isl], srows_v.at[b], sg.at[b])

  def gwait(b):
    pltpu.make_async_copy(
        tbl.at[pl.ds(0, _CHUNK_ROWS)], rows_v.at[b], sg.at[b]).wait()
    if with_len:
      pltpu.make_async_copy(
          s_hbm.at[pl.ds(0, _CHUNK_ROWS)], srows_v.at[b], sg.at[b]).wait()

  def wstart(j, b):
    pltpu.async_copy(
        out_v.at[b],
        sumsh.at[pl.ds(base + j * _NODES_PER_CHUNK, _NODES_PER_CHUNK)],
        sw.at[b])

  def wwait(b):
    pltpu.make_async_copy(
        out_v.at[b], sumsh.at[pl.ds(0, _NODES_PER_CHUNK)], sw.at[b]).wait()

  gstart(0, 0)
  gstart(1, 1)

  def body(j, _):
    b = j % 2
    gwait(b)

    @pl.when(j >= 2)
    def _():
      wwait(b)

    _reduce_chunk(rows_v, srows_v, b, j, out_v, len_v, with_len)
    wstart(j, b)

    @pl.when(j + 2 < _N_CHUNKS)
    def _():
      gstart(j + 2, b)

    return 0

  lax.fori_loop(0, _N_CHUNKS, body, 0)
  wwait(0)
  wwait(1)
  if with_len:
    pltpu.sync_copy(len_v, lenh.at[pl.ds(base, ROWS_PER_W)])


def _neigh_sum_body(with_len, *refs):
  if with_len:
    (table_hbm, idxf_hbm, idxb_hbm, s_hbm,
     sumsf_hbm, sumsb_hbm, lenf_hbm, lenb_hbm,
     idx_v, rows_v, srows_v, out_v, len_v, sg, sw) = refs
  else:
    (tablef_hbm, tableb_hbm, idxf_hbm, idxb_hbm,
     sumsf_hbm, sumsb_hbm,
     idx_v, rows_v, out_v, sg, sw) = refs
    srows_v = len_v = s_hbm = None

  wid = lax.axis_index("s") * NC + lax.axis_index("c")
  base = wid * ROWS_PER_W

  if with_len:
    _neigh_phase(table_hbm, idxf_hbm, s_hbm, sumsf_hbm, lenf_hbm, wid, base,
                 idx_v, rows_v, srows_v, out_v, len_v, sg, sw, True)
    _neigh_phase(table_hbm, idxb_hbm, s_hbm, sumsb_hbm, lenb_hbm, wid, base,
                 idx_v, rows_v, srows_v, out_v, len_v, sg, sw, True)
  else:
    _neigh_phase(tablef_hbm, idxf_hbm, None, sumsf_hbm, None, wid, base,
                 idx_v, rows_v, None, out_v, None, sg, sw, False)
    _neigh_phase(tableb_hbm, idxb_hbm, None, sumsb_hbm, None, wid, base,
                 idx_v, rows_v, None, out_v, None, sg, sw, False)


def _neigh_sum_pair(table_f, table_b, idx_f, idx_b, s128=None):
  with_len = s128 is not None
  scratch = [
      pltpu.VMEM((64, 128), jnp.int32),
      pltpu.VMEM((2, _CHUNK_ROWS, EMB), jnp.float32),
  ]
  if with_len:
    out_type = (jax.ShapeDtypeStruct((N_NODES, EMB), jnp.float32),
                jax.ShapeDtypeStruct((N_NODES, EMB), jnp.float32),
                jax.ShapeDtypeStruct((N_NODES, 16), jnp.float32),
                jax.ShapeDtypeStruct((N_NODES, 16), jnp.float32))
    scratch.append(pltpu.VMEM((2, _CHUNK_ROWS, EMB), jnp.float32))
    args = [table_f, idx_f, idx_b, s128]
  else:
    out_type = (jax.ShapeDtypeStruct((N_NODES, EMB), jnp.float32),
                jax.ShapeDtypeStruct((N_NODES, EMB), jnp.float32))
    args = [table_f, table_b, idx_f, idx_b]
  scratch.append(pltpu.VMEM((2, _NODES_PER_CHUNK, EMB), jnp.float32))
  if with_len:
    scratch.append(pltpu.VMEM((ROWS_PER_W, 16), jnp.float32))
  scratch.append(pltpu.SemaphoreType.DMA((2,)))
  scratch.append(pltpu.SemaphoreType.DMA((2,)))
  f = pl.kernel(
      functools.partial(_neigh_sum_body, with_len),
      out_type=out_type,
      mesh=_sc_mesh(),
      scratch_types=scratch,
  )
  return f(*args)


_TBLK = 64
_NGRID = SEQ // _TBLK


def _lstm_body(xsf_ref, xsb_ref, wfT_ref, wbT_ref, bf_ref, bb_ref, wblk_ref,
               ysf_ref, ysb_ref, hf, cf, hb, cb, gf_s, gb_s):
  i = pl.program_id(0)

  @pl.when(i == 0)
  def _():
    hf[...] = jnp.zeros((BATCH, H_DIR), jnp.float32)
    cf[...] = jnp.zeros((BATCH, H_DIR), jnp.float32)
    hb[...] = jnp.zeros((BATCH, H_DIR), jnp.float32)
    cb[...] = jnp.zeros((BATCH, H_DIR), jnp.float32)

  xf = xsf_ref[...].reshape(_TBLK * BATCH, EMB)
  gf_s[...] = (jnp.dot(xf, wfT_ref[...], preferred_element_type=jnp.float32)
               + bf_ref[0:1, :]).reshape(_TBLK, BATCH, 4 * H_DIR)
  xb = xsb_ref[...].reshape(_TBLK * BATCH, EMB)
  gb_s[...] = (jnp.dot(xb, wbT_ref[...], preferred_element_type=jnp.float32)
               + bb_ref[0:1, :]).reshape(_TBLK, BATCH, 4 * H_DIR)

  def step(k, _):
    tb = _TBLK - 1 - k
    hcat = jnp.concatenate([hf[...], hb[...]], axis=1)
    g2 = jnp.dot(hcat, wblk_ref[...], preferred_element_type=jnp.float32)
    gfk = gf_s[k] + g2[:, :4 * H_DIR]
    gbk = gb_s[tb] + g2[:, 4 * H_DIR:]
    for g, h_r, c_r, ys_r, t in ((gfk, hf, cf, ysf_ref, k),
                                 (gbk, hb, cb, ysb_ref, tb)):
      ig = jax.nn.sigmoid(g[:, :H_DIR])
      fg = jax.nn.sigmoid(g[:, H_DIR:2 * H_DIR])
      gg = jnp.tanh(g[:, 2 * H_DIR:3 * H_DIR])
      og = jax.nn.sigmoid(g[:, 3 * H_DIR:])
      c2 = fg * c_r[...] + ig * gg
      h2 = og * jnp.tanh(c2)
      c_r[...] = c2
      h_r[...] = h2
      ys_r[t] = h2
    return 0

  lax.fori_loop(0, _TBLK, step, 0)


def _lstm_layer(xs, W_ih, W_hh, b_ih, b_hh, layer):
  wfT = W_ih[layer, 0].T
  wbT = W_ih[layer, 1].T
  bf = jnp.tile((b_ih[layer, 0] + b_hh[layer, 0])[None, :], (8, 1))
  bb = jnp.tile((b_ih[layer, 1] + b_hh[layer, 1])[None, :], (8, 1))
  wblk = jnp.zeros((2 * H_DIR, 8 * H_DIR), jnp.float32)
  wblk = wblk.at[:H_DIR, :4 * H_DIR].set(W_hh[layer, 0].T)
  wblk = wblk.at[H_DIR:, 4 * H_DIR:].set(W_hh[layer, 1].T)

  grid = (_NGRID,)
  blk = pl.BlockSpec((_TBLK, BATCH, EMB), lambda i: (i, 0, 0))
  blk_rev = pl.BlockSpec((_TBLK, BATCH, EMB), lambda i: (_NGRID - 1 - i, 0, 0))
  full = lambda shape: pl.BlockSpec(shape, lambda i: tuple(0 for _ in shape))
  oblk = pl.BlockSpec((_TBLK, BATCH, H_DIR), lambda i: (i, 0, 0))
  oblk_rev = pl.BlockSpec((_TBLK, BATCH, H_DIR),
                          lambda i: (_NGRID - 1 - i, 0, 0))
  return pl.pallas_call(
      _lstm_body,
      grid=grid,
      in_specs=[blk, blk_rev, full((EMB, 4 * H_DIR)), full((EMB, 4 * H_DIR)),
                full((8, 4 * H_DIR)), full((8, 4 * H_DIR)),
                full((2 * H_DIR, 8 * H_DIR))],
      out_specs=[oblk, oblk_rev],
      out_shape=[jax.ShapeDtypeStruct((SEQ, BATCH, H_DIR), jnp.float32),
                 jax.ShapeDtypeStruct((SEQ, BATCH, H_DIR), jnp.float32)],
      scratch_shapes=[pltpu.VMEM((BATCH, H_DIR), jnp.float32)] * 4
      + [pltpu.VMEM((_TBLK, BATCH, 4 * H_DIR), jnp.float32)] * 2,
      compiler_params=pltpu.CompilerParams(
          dimension_semantics=("arbitrary",)),
  )(xs, xs, wfT, wbT, bf, bb, wblk)


def _row_sign_body(x_ref, ones_ref, out_ref):
  r = jnp.dot(jax.nn.relu(x_ref[...]), ones_ref[...],
              preferred_element_type=jnp.float32)
  out_ref[...] = (r > 0.0).astype(jnp.float32)


def _row_sign(x):
  ones = jnp.ones((EMB, 8), jnp.float32)
  return pl.pallas_call(
      _row_sign_body,
      grid=(32,),
      in_specs=[pl.BlockSpec((512, EMB), lambda i: (i, 0)),
                pl.BlockSpec((EMB, 8), lambda i: (0, 0))],
      out_specs=pl.BlockSpec((512, 8), lambda i: (i, 0)),
      out_shape=jax.ShapeDtypeStruct((N_NODES, 8), jnp.float32),
  )(x, ones)


def _agg_body(h_ref, sums_ref, len_ref, w_ref, b_ref, out_ref):
  recip = 1.0 / jnp.maximum(len_ref[:, 0:1], 1.0)
  means = sums_ref[...] * recip
  g = (jnp.dot(h_ref[...], w_ref[:HIDDEN, :],
               preferred_element_type=jnp.float32)
       + jnp.dot(means, w_ref[HIDDEN:, :], preferred_element_type=jnp.float32)
       + b_ref[0:1, :])
  out_ref[...] = jnp.maximum(g, 0.0)


def _agg(h, sums, len16, W, b):
  b8 = jnp.tile(b[None, :], (8, 1))
  return pl.pallas_call(
      _agg_body,
      grid=(32,),
      in_specs=[pl.BlockSpec((512, HIDDEN), lambda i: (i, 0)),
                pl.BlockSpec((512, HIDDEN), lambda i: (i, 0)),
                pl.BlockSpec((512, 16), lambda i: (i, 0)),
                pl.BlockSpec((2 * HIDDEN, HIDDEN), lambda i: (0, 0)),
                pl.BlockSpec((8, HIDDEN), lambda i: (0, 0))],
      out_specs=pl.BlockSpec((512, HIDDEN), lambda i: (i, 0)),
      out_shape=jax.ShapeDtypeStruct((N_NODES, HIDDEN), jnp.float32),
  )(h, sums, len16, W, b8)


def kernel(fw_adj_info, bw_adj_info, feature_info, batch_nodes, embedding,
           lstm_W_ih, lstm_W_hh, lstm_b_ih, lstm_b_hh, padding_vector,
           fw_agg_W, fw_agg_b, bw_agg_W, bw_agg_b):
  bsz, seq = batch_nodes.shape

  feat2d = feature_info[:-1].reshape(-1).astype(jnp.int32).reshape(128, 128)
  x = _emb_gather(embedding, feat2d)

  xs = x.reshape(bsz, seq, EMB).transpose(1, 0, 2)
  for layer in range(2):
    ysf, ysb = _lstm_layer(xs, lstm_W_ih, lstm_W_hh, lstm_b_ih, lstm_b_hh,
                           layer)
    xs = jnp.concatenate([ysf, ysb], axis=-1)
  feature_vector = xs.transpose(1, 0, 2).reshape(N_NODES, HIDDEN)

  node_repres = jnp.concatenate([feature_vector, padding_vector], axis=0)

  s8 = _row_sign(feature_vector)
  s_pad = (jnp.sum(jax.nn.relu(padding_vector)) > 0.0).astype(jnp.float32)
  s_col = jnp.concatenate([s8[:, :1], s_pad.reshape(1, 1)], axis=0)
  s128 = jnp.pad(s_col, ((0, 0), (0, EMB - 1)))

  fw_idx = fw_adj_info.astype(jnp.int32).reshape(NW * 64, 128)
  bw_idx = bw_adj_info.astype(jnp.int32).reshape(NW * 64, 128)

  sums_f, sums_b, len_f, len_b = _neigh_sum_pair(
      node_repres, None, fw_idx, bw_idx, s128)

  fw_h = _agg(feature_vector, sums_f, len_f, fw_agg_W[0], fw_agg_b[0])
  bw_h = _agg(feature_vector, sums_b, len_b, bw_agg_W[0], bw_agg_b[0])

  zero_row = jnp.zeros((1, HIDDEN), jnp.float32)
  for layer in range(1, N_LAYERS):
    tf = jnp.concatenate([fw_h, zero_row], axis=0)
    tb = jnp.concatenate([bw_h, zero_row], axis=0)
    sums_f, sums_b = _neigh_sum_pair(tf, tb, fw_idx, bw_idx)
    fw_h = _agg(fw_h, sums_f, len_f, fw_agg_W[layer], fw_agg_b[layer])
    bw_h = _agg(bw_h, sums_b, len_b, bw_agg_W[layer], bw_agg_b[layer])

  return jnp.concatenate([fw_h.reshape(bsz, seq, HIDDEN),
                          bw_h.reshape(bsz, seq, HIDDEN)], axis=-1)


# speedup vs baseline: 7.9740x; 1.3276x over previous
import functools

import jax
import jax.numpy as jnp
from jax import lax
from jax.experimental import pallas as pl
from jax.experimental.pallas import tpu as pltpu
from jax.experimental.pallas import tpu_sc as plsc

HIDDEN = 128
H_DIR = 64
SAMPLE = 16
N_LAYERS = 3
N_NODES = 16384
BATCH = 16
SEQ = 1024
EMB = 128

NC = 2
NS = 16
NW = NC * NS
ROWS_PER_W = N_NODES // NW

_sc_mesh_cache = []


def _sc_mesh():
  if not _sc_mesh_cache:
    _sc_mesh_cache.append(plsc.VectorSubcoreMesh(
        core_axis_name="c", subcore_axis_name="s",
        num_cores=NC, num_subcores=NS))
  return _sc_mesh_cache[0]


_EMB_CHUNKS = 8


def _emb_gather_body(table_hbm, idx_hbm, out_hbm, idx_v, rows_v, sg, sw):
  wid = lax.axis_index("s") * NC + lax.axis_index("c")
  base = wid * ROWS_PER_W

  pltpu.sync_copy(idx_hbm.at[pl.ds(wid * 4, 4)], idx_v)

  def gstart(j, b):
    isl = idx_v.at[j // 2, pl.ds((j % 2) * 64, 64)]
    pltpu.async_copy(table_hbm.at[isl], rows_v.at[b], sg.at[b])

  def gwait(b):
    pltpu.make_async_copy(
        table_hbm.at[pl.ds(0, 64)], rows_v.at[b], sg.at[b]).wait()

  def wstart(j, b):
    pltpu.async_copy(rows_v.at[b], out_hbm.at[pl.ds(base + j * 64, 64)],
                     sw.at[b])

  def wwait(b):
    pltpu.make_async_copy(rows_v.at[b], out_hbm.at[pl.ds(0, 64)],
                          sw.at[b]).wait()

  gstart(0, 0)
  gstart(1, 1)

  def body(j, _):
    b = j % 2
    gwait(b)
    wstart(j, b)

    @pl.when(j + 2 < _EMB_CHUNKS)
    def _():
      wwait(b)
      gstart(j + 2, b)

    return 0

  lax.fori_loop(0, _EMB_CHUNKS, body, 0)
  wwait(0)
  wwait(1)


def _emb_gather(table, idx2d):
  f = pl.kernel(
      _emb_gather_body,
      out_type=jax.ShapeDtypeStruct((N_NODES, EMB), jnp.float32),
      mesh=_sc_mesh(),
      scratch_types=[
          pltpu.VMEM((4, 128), jnp.int32),
          pltpu.VMEM((2, 64, EMB), jnp.float32),
          pltpu.SemaphoreType.DMA((2,)),
          pltpu.SemaphoreType.DMA((2,)),
      ],
  )
  return f(table, idx2d)


_N_CHUNKS = 256
_CHUNK_ROWS = 32
_NODES_PER_CHUNK = 2
_NBUF = 4


def _reduce_chunk(rows_v, srows_v, b, j, out_v, len_v, with_len):
  def node_body(k, _):
    rbase = k * SAMPLE
    for cg in range(EMB // 16):
      acc = rows_v[b, rbase, pl.ds(cg * 16, 16)]
      for r in range(1, SAMPLE):
        acc = acc + rows_v[b, rbase + r, pl.ds(cg * 16, 16)]
      out_v[b, k, pl.ds(cg * 16, 16)] = acc
    if with_len:
      accl = srows_v[b, rbase, pl.ds(0, 16)]
      for r in range(1, SAMPLE):
        accl = accl + srows_v[b, rbase + r, pl.ds(0, 16)]
      len_v[j * _NODES_PER_CHUNK + k, :] = accl
    return 0

  lax.fori_loop(0, _NODES_PER_CHUNK, node_body, 0)


def _neigh_phase(tbl, idxh, s_hbm, sumsh, lenh, wid, base,
                 idx_v, rows_v, srows_v, out_v, len_v, sg, sw, with_len):
  pltpu.sync_copy(idxh.at[pl.ds(wid * 64, 64)], idx_v)

  def gstart(j, b):
    isl = idx_v.at[j // 4, pl.ds((j % 4) * 32, 32)]
    pltpu.async_copy(tbl.at[isl], rows_v.at[b], sg.at[b])
    if with_len:
      pltpu.async_copy(s_hbm.at[isl], srows_v.at[b], sg.at[b])

  def gwait(b):
    pltpu.make_async_copy(
        tbl.at[pl.ds(0, _CHUNK_ROWS)], rows_v.at[b], sg.at[b]).wait()
    if with_len:
      pltpu.make_async_copy(
          s_hbm.at[pl.ds(0, _CHUNK_ROWS)], srows_v.at[b], sg.at[b]).wait()

  def wstart(j, b):
    pltpu.async_copy(
        out_v.at[b],
        sumsh.at[pl.ds(base + j * _NODES_PER_CHUNK, _NODES_PER_CHUNK)],
        sw.at[b])

  def wwait(b):
    pltpu.make_async_copy(
        out_v.at[b], sumsh.at[pl.ds(0, _NODES_PER_CHUNK)], sw.at[b]).wait()

  for jj in range(_NBUF):
    gstart(jj, jj)

  def body(j, _):
    b = j % _NBUF
    gwait(b)

    @pl.when(j >= _NBUF)
    def _():
      wwait(b)

    _reduce_chunk(rows_v, srows_v, b, j, out_v, len_v, with_len)
    wstart(j, b)

    @pl.when(j + _NBUF < _N_CHUNKS)
    def _():
      gstart(j + _NBUF, b)

    return 0

  lax.fori_loop(0, _N_CHUNKS, body, 0)
  for jj in range(_NBUF):
    wwait(jj)
  if with_len:
    pltpu.sync_copy(len_v, lenh.at[pl.ds(base, ROWS_PER_W)])


def _neigh_sum_body(with_len, *refs):
  if with_len:
    (table_hbm, idxf_hbm, idxb_hbm, s_hbm,
     sumsf_hbm, sumsb_hbm, lenf_hbm, lenb_hbm,
     idx_v, rows_v, srows_v, out_v, len_v, sg, sw) = refs
  else:
    (tablef_hbm, tableb_hbm, idxf_hbm, idxb_hbm,
     sumsf_hbm, sumsb_hbm,
     idx_v, rows_v, out_v, sg, sw) = refs
    srows_v = len_v = s_hbm = None

  wid = lax.axis_index("s") * NC + lax.axis_index("c")
  base = wid * ROWS_PER_W

  if with_len:
    _neigh_phase(table_hbm, idxf_hbm, s_hbm, sumsf_hbm, lenf_hbm, wid, base,
                 idx_v, rows_v, srows_v, out_v, len_v, sg, sw, True)
    _neigh_phase(table_hbm, idxb_hbm, s_hbm, sumsb_hbm, lenb_hbm, wid, base,
                 idx_v, rows_v, srows_v, out_v, len_v, sg, sw, True)
  else:
    _neigh_phase(tablef_hbm, idxf_hbm, None, sumsf_hbm, None, wid, base,
                 idx_v, rows_v, None, out_v, None, sg, sw, False)
    _neigh_phase(tableb_hbm, idxb_hbm, None, sumsb_hbm, None, wid, base,
                 idx_v, rows_v, None, out_v, None, sg, sw, False)


def _neigh_sum_pair(table_f, table_b, idx_f, idx_b, s128=None):
  with_len = s128 is not None
  scratch = [
      pltpu.VMEM((64, 128), jnp.int32),
      pltpu.VMEM((_NBUF, _CHUNK_ROWS, EMB), jnp.float32),
  ]
  if with_len:
    out_type = (jax.ShapeDtypeStruct((N_NODES, EMB), jnp.float32),
                jax.ShapeDtypeStruct((N_NODES, EMB), jnp.float32),
                jax.ShapeDtypeStruct((N_NODES, 16), jnp.float32),
                jax.ShapeDtypeStruct((N_NODES, 16), jnp.float32))
    scratch.append(pltpu.VMEM((_NBUF, _CHUNK_ROWS, EMB), jnp.float32))
    args = [table_f, idx_f, idx_b, s128]
  else:
    out_type = (jax.ShapeDtypeStruct((N_NODES, EMB), jnp.float32),
                jax.ShapeDtypeStruct((N_NODES, EMB), jnp.float32))
    args = [table_f, table_b, idx_f, idx_b]
  scratch.append(pltpu.VMEM((_NBUF, _NODES_PER_CHUNK, EMB), jnp.float32))
  if with_len:
    scratch.append(pltpu.VMEM((ROWS_PER_W, 16), jnp.float32))
  scratch.append(pltpu.SemaphoreType.DMA((_NBUF,)))
  scratch.append(pltpu.SemaphoreType.DMA((_NBUF,)))
  f = pl.kernel(
      functools.partial(_neigh_sum_body, with_len),
      out_type=out_type,
      mesh=_sc_mesh(),
      scratch_types=scratch,
  )
  return f(*args)


_TBLK = 64
_NGRID = SEQ // _TBLK


def _lstm_body(xsf_ref, xsb_ref, wfT_ref, wbT_ref, bf_ref, bb_ref, wfh_ref, wbh_ref,
               ysf_ref, ysb_ref, h_s, c_s, gf_s, gb_s):
  i = pl.program_id(0)

  @pl.when(i == 0)
  def _():
    h_s[...] = jnp.zeros((BATCH, 2 * H_DIR), jnp.float32)
    c_s[...] = jnp.zeros((BATCH, 2 * H_DIR), jnp.float32)

  xf = xsf_ref[...].reshape(_TBLK * BATCH, EMB)
  gf_s[...] = (jnp.dot(xf, wfT_ref[...], preferred_element_type=jnp.float32)
               + bf_ref[0:1, :]).reshape(_TBLK, BATCH, 4 * H_DIR)
  xb = xsb_ref[...].reshape(_TBLK * BATCH, EMB)
  gb_s[...] = (jnp.dot(xb, wbT_ref[...], preferred_element_type=jnp.float32)
               + bb_ref[0:1, :]).reshape(_TBLK, BATCH, 4 * H_DIR)

  def step(k, hc):
    hf, hb, cf, cb = hc
    tb = _TBLK - 1 - k
    gfk = gf_s[k] + jnp.dot(hf, wfh_ref[...],
                            preferred_element_type=jnp.float32,
                            precision=lax.Precision.DEFAULT)
    gbk = gb_s[tb] + jnp.dot(hb, wbh_ref[...],
                             preferred_element_type=jnp.float32,
                             precision=lax.Precision.DEFAULT)
    out = []
    for g, c_prev, ys_r, t in ((gfk, cf, ysf_ref, k),
                               (gbk, cb, ysb_ref, tb)):
      ig = jax.nn.sigmoid(g[:, :H_DIR])
      fg = jax.nn.sigmoid(g[:, H_DIR:2 * H_DIR])
      gg = jnp.tanh(g[:, 2 * H_DIR:3 * H_DIR])
      og = jax.nn.sigmoid(g[:, 3 * H_DIR:])
      c2 = fg * c_prev + ig * gg
      h2 = og * jnp.tanh(c2)
      out.append((h2, c2))
      ys_r[t] = h2
    return (out[0][0], out[1][0], out[0][1], out[1][1])

  hc = lax.fori_loop(0, _TBLK, step,
                     (h_s[:, :H_DIR], h_s[:, H_DIR:],
                      c_s[:, :H_DIR], c_s[:, H_DIR:]), unroll=4)
  h_s[:, :H_DIR] = hc[0]
  h_s[:, H_DIR:] = hc[1]
  c_s[:, :H_DIR] = hc[2]
  c_s[:, H_DIR:] = hc[3]


def _lstm_layer(xs, W_ih, W_hh, b_ih, b_hh, layer):
  wfT = W_ih[layer, 0].T
  wbT = W_ih[layer, 1].T
  bf = jnp.tile((b_ih[layer, 0] + b_hh[layer, 0])[None, :], (8, 1))
  bb = jnp.tile((b_ih[layer, 1] + b_hh[layer, 1])[None, :], (8, 1))
  wfh = W_hh[layer, 0].T
  wbh = W_hh[layer, 1].T

  grid = (_NGRID,)
  blk = pl.BlockSpec((_TBLK, BATCH, EMB), lambda i: (i, 0, 0))
  blk_rev = pl.BlockSpec((_TBLK, BATCH, EMB), lambda i: (_NGRID - 1 - i, 0, 0))
  full = lambda shape: pl.BlockSpec(shape, lambda i: tuple(0 for _ in shape))
  oblk = pl.BlockSpec((_TBLK, BATCH, H_DIR), lambda i: (i, 0, 0))
  oblk_rev = pl.BlockSpec((_TBLK, BATCH, H_DIR),
                          lambda i: (_NGRID - 1 - i, 0, 0))
  return pl.pallas_call(
      _lstm_body,
      grid=grid,
      in_specs=[blk, blk_rev, full((EMB, 4 * H_DIR)), full((EMB, 4 * H_DIR)),
                full((8, 4 * H_DIR)), full((8, 4 * H_DIR)),
                full((H_DIR, 4 * H_DIR)), full((H_DIR, 4 * H_DIR))],
      out_specs=[oblk, oblk_rev],
      out_shape=[jax.ShapeDtypeStruct((SEQ, BATCH, H_DIR), jnp.float32),
                 jax.ShapeDtypeStruct((SEQ, BATCH, H_DIR), jnp.float32)],
      scratch_shapes=[pltpu.VMEM((BATCH, 2 * H_DIR), jnp.float32)] * 2
      + [pltpu.VMEM((_TBLK, BATCH, 4 * H_DIR), jnp.float32)] * 2,
      compiler_params=pltpu.CompilerParams(
          dimension_semantics=("arbitrary",)),
  )(xs, xs, wfT, wbT, bf, bb, wfh, wbh)


def _row_sign_body(x_ref, ones_ref, out_ref):
  r = jnp.dot(jax.nn.relu(x_ref[...]), ones_ref[...],
              preferred_element_type=jnp.float32)
  out_ref[...] = (r > 0.0).astype(jnp.float32)


def _row_sign(x):
  ones = jnp.ones((EMB, 8), jnp.float32)
  return pl.pallas_call(
      _row_sign_body,
      grid=(32,),
      in_specs=[pl.BlockSpec((512, EMB), lambda i: (i, 0)),
                pl.BlockSpec((EMB, 8), lambda i: (0, 0))],
      out_specs=pl.BlockSpec((512, 8), lambda i: (i, 0)),
      out_shape=jax.ShapeDtypeStruct((N_NODES, 8), jnp.float32),
  )(x, ones)


def _agg_body(h_ref, sums_ref, len_ref, w_ref, b_ref, out_ref):
  recip = 1.0 / jnp.maximum(len_ref[:, 0:1], 1.0)
  means = sums_ref[...] * recip
  g = (jnp.dot(h_ref[...], w_ref[:HIDDEN, :],
               preferred_element_type=jnp.float32)
       + jnp.dot(means, w_ref[HIDDEN:, :], preferred_element_type=jnp.float32)
       + b_ref[0:1, :])
  out_ref[...] = jnp.maximum(g, 0.0)


def _agg(h, sums, len16, W, b):
  b8 = jnp.tile(b[None, :], (8, 1))
  return pl.pallas_call(
      _agg_body,
      grid=(32,),
      in_specs=[pl.BlockSpec((512, HIDDEN), lambda i: (i, 0)),
                pl.BlockSpec((512, HIDDEN), lambda i: (i, 0)),
                pl.BlockSpec((512, 16), lambda i: (i, 0)),
                pl.BlockSpec((2 * HIDDEN, HIDDEN), lambda i: (0, 0)),
                pl.BlockSpec((8, HIDDEN), lambda i: (0, 0))],
      out_specs=pl.BlockSpec((512, HIDDEN), lambda i: (i, 0)),
      out_shape=jax.ShapeDtypeStruct((N_NODES, HIDDEN), jnp.float32),
  )(h, sums, len16, W, b8)


def kernel(fw_adj_info, bw_adj_info, feature_info, batch_nodes, embedding,
           lstm_W_ih, lstm_W_hh, lstm_b_ih, lstm_b_hh, padding_vector,
           fw_agg_W, fw_agg_b, bw_agg_W, bw_agg_b):
  bsz, seq = batch_nodes.shape

  feat2d = feature_info[:-1].reshape(-1).astype(jnp.int32).reshape(128, 128)
  x = _emb_gather(embedding, feat2d)

  xs = x.reshape(bsz, seq, EMB).transpose(1, 0, 2)
  for layer in range(2):
    ysf, ysb = _lstm_layer(xs, lstm_W_ih, lstm_W_hh, lstm_b_ih, lstm_b_hh,
                           layer)
    xs = jnp.concatenate([ysf, ysb], axis=-1)
  feature_vector = xs.transpose(1, 0, 2).reshape(N_NODES, HIDDEN)

  node_repres = jnp.concatenate([feature_vector, padding_vector], axis=0)

  s8 = _row_sign(feature_vector)
  s_pad = (jnp.sum(jax.nn.relu(padding_vector)) > 0.0).astype(jnp.float32)
  s_col = jnp.concatenate([s8[:, :1], s_pad.reshape(1, 1)], axis=0)
  s128 = jnp.pad(s_col, ((0, 0), (0, EMB - 1)))

  fw_idx = fw_adj_info.astype(jnp.int32).reshape(NW * 64, 128)
  bw_idx = bw_adj_info.astype(jnp.int32).reshape(NW * 64, 128)

  sums_f, sums_b, len_f, len_b = _neigh_sum_pair(
      node_repres, None, fw_idx, bw_idx, s128)

  fw_h = _agg(feature_vector, sums_f, len_f, fw_agg_W[0], fw_agg_b[0])
  bw_h = _agg(feature_vector, sums_b, len_b, bw_agg_W[0], bw_agg_b[0])

  zero_row = jnp.zeros((1, HIDDEN), jnp.float32)
  for layer in range(1, N_LAYERS):
    tf = jnp.concatenate([fw_h, zero_row], axis=0)
    tb = jnp.concatenate([bw_h, zero_row], axis=0)
    sums_f, sums_b = _neigh_sum_pair(tf, tb, fw_idx, bw_idx)
    fw_h = _agg(fw_h, sums_f, len_f, fw_agg_W[layer], fw_agg_b[layer])
    bw_h = _agg(bw_h, sums_b, len_b, bw_agg_W[layer], bw_agg_b[layer])

  return jnp.concatenate([fw_h.reshape(bsz, seq, HIDDEN),
                          bw_h.reshape(bsz, seq, HIDDEN)], axis=-1)


# speedup vs baseline: 8.0988x; 1.0156x over previous
import functools

import jax
import jax.numpy as jnp
from jax import lax
from jax.experimental import pallas as pl
from jax.experimental.pallas import tpu as pltpu
from jax.experimental.pallas import tpu_sc as plsc

HIDDEN = 128
H_DIR = 64
SAMPLE = 16
N_LAYERS = 3
N_NODES = 16384
BATCH = 16
SEQ = 1024
EMB = 128

NC = 2
NS = 16
NW = NC * NS
ROWS_PER_W = N_NODES // NW

_sc_mesh_cache = []


def _sc_mesh():
  if not _sc_mesh_cache:
    _sc_mesh_cache.append(plsc.VectorSubcoreMesh(
        core_axis_name="c", subcore_axis_name="s",
        num_cores=NC, num_subcores=NS))
  return _sc_mesh_cache[0]


_EMB_CHUNKS = 8


def _emb_gather_body(table_hbm, idx_hbm, out_hbm, idx_v, rows_v, sg, sw):
  wid = lax.axis_index("s") * NC + lax.axis_index("c")
  base = wid * ROWS_PER_W

  pltpu.sync_copy(idx_hbm.at[pl.ds(wid * 4, 4)], idx_v)

  def gstart(j, b):
    isl = idx_v.at[j // 2, pl.ds((j % 2) * 64, 64)]
    pltpu.async_copy(table_hbm.at[isl], rows_v.at[b], sg.at[b])

  def gwait(b):
    pltpu.make_async_copy(
        table_hbm.at[pl.ds(0, 64)], rows_v.at[b], sg.at[b]).wait()

  def wstart(j, b):
    pltpu.async_copy(rows_v.at[b], out_hbm.at[pl.ds(base + j * 64, 64)],
                     sw.at[b])

  def wwait(b):
    pltpu.make_async_copy(rows_v.at[b], out_hbm.at[pl.ds(0, 64)],
                          sw.at[b]).wait()

  gstart(0, 0)
  gstart(1, 1)

  def body(j, _):
    b = j % 2
    gwait(b)
    wstart(j, b)

    @pl.when(j + 2 < _EMB_CHUNKS)
    def _():
      wwait(b)
      gstart(j + 2, b)

    return 0

  lax.fori_loop(0, _EMB_CHUNKS, body, 0)
  wwait(0)
  wwait(1)


def _emb_gather(table, idx2d):
  f = pl.kernel(
      _emb_gather_body,
      out_type=jax.ShapeDtypeStruct((N_NODES, EMB), jnp.float32),
      mesh=_sc_mesh(),
      scratch_types=[
          pltpu.VMEM((4, 128), jnp.int32),
          pltpu.VMEM((2, 64, EMB), jnp.float32),
          pltpu.SemaphoreType.DMA((2,)),
          pltpu.SemaphoreType.DMA((2,)),
      ],
  )
  return f(table, idx2d)


_N_CHUNKS = 256
_CHUNK_ROWS = 32
_NODES_PER_CHUNK = 2
_NBUF = 4


def _reduce_chunk(rows_v, srows_v, b, j, out_v, len_v, with_len):
  def node_body(k, _):
    rbase = k * SAMPLE
    for cg in range(EMB // 16):
      acc = rows_v[b, rbase, pl.ds(cg * 16, 16)]
      for r in range(1, SAMPLE):
        acc = acc + rows_v[b, rbase + r, pl.ds(cg * 16, 16)]
      out_v[b, k, pl.ds(cg * 16, 16)] = acc
    if with_len:
      accl = srows_v[b, rbase, pl.ds(0, 16)]
      for r in range(1, SAMPLE):
        accl = accl + srows_v[b, rbase + r, pl.ds(0, 16)]
      len_v[j * _NODES_PER_CHUNK + k, :] = accl
    return 0

  lax.fori_loop(0, _NODES_PER_CHUNK, node_body, 0)


def _neigh_phase(tbl, idxh, s_hbm, sumsh, lenh, wid, base,
                 idx_v, rows_v, srows_v, out_v, len_v, sg, sw, with_len):
  pltpu.sync_copy(idxh.at[pl.ds(wid * 64, 64)], idx_v)

  def gstart(j, b):
    isl = idx_v.at[j // 4, pl.ds((j % 4) * 32, 32)]
    pltpu.async_copy(tbl.at[isl], rows_v.at[b], sg.at[b])
    if with_len:
      pltpu.async_copy(s_hbm.at[isl], srows_v.at[b], sg.at[b])

  def gwait(b):
    pltpu.make_async_copy(
        tbl.at[pl.ds(0, _CHUNK_ROWS)], rows_v.at[b], sg.at[b]).wait()
    if with_len:
      pltpu.make_async_copy(
          s_hbm.at[pl.ds(0, _CHUNK_ROWS)], srows_v.at[b], sg.at[b]).wait()

  def wstart(j, b):
    pltpu.async_copy(
        out_v.at[b],
        sumsh.at[pl.ds(base + j * _NODES_PER_CHUNK, _NODES_PER_CHUNK)],
        sw.at[b])

  def wwait(b):
    pltpu.make_async_copy(
        out_v.at[b], sumsh.at[pl.ds(0, _NODES_PER_CHUNK)], sw.at[b]).wait()

  for jj in range(_NBUF):
    gstart(jj, jj)

  def body(j, _):
    b = j % _NBUF
    gwait(b)

    @pl.when(j >= _NBUF)
    def _():
      wwait(b)

    _reduce_chunk(rows_v, srows_v, b, j, out_v, len_v, with_len)
    wstart(j, b)

    @pl.when(j + _NBUF < _N_CHUNKS)
    def _():
      gstart(j + _NBUF, b)

    return 0

  lax.fori_loop(0, _N_CHUNKS, body, 0)
  for jj in range(_NBUF):
    wwait(jj)
  if with_len:
    pltpu.sync_copy(len_v, lenh.at[pl.ds(base, ROWS_PER_W)])


def _neigh_sum_body(with_len, *refs):
  if with_len:
    (table_hbm, idxf_hbm, idxb_hbm, s_hbm,
     sumsf_hbm, sumsb_hbm, lenf_hbm, lenb_hbm,
     idx_v, rows_v, srows_v, out_v, len_v, sg, sw) = refs
  else:
    (tablef_hbm, tableb_hbm, idxf_hbm, idxb_hbm,
     sumsf_hbm, sumsb_hbm,
     idx_v, rows_v, out_v, sg, sw) = refs
    srows_v = len_v = s_hbm = None

  wid = lax.axis_index("s") * NC + lax.axis_index("c")
  base = wid * ROWS_PER_W

  if with_len:
    _neigh_phase(table_hbm, idxf_hbm, s_hbm, sumsf_hbm, lenf_hbm, wid, base,
                 idx_v, rows_v, srows_v, out_v, len_v, sg, sw, True)
    _neigh_phase(table_hbm, idxb_hbm, s_hbm, sumsb_hbm, lenb_hbm, wid, base,
                 idx_v, rows_v, srows_v, out_v, len_v, sg, sw, True)
  else:
    _neigh_phase(tablef_hbm, idxf_hbm, None, sumsf_hbm, None, wid, base,
                 idx_v, rows_v, None, out_v, None, sg, sw, False)
    _neigh_phase(tableb_hbm, idxb_hbm, None, sumsb_hbm, None, wid, base,
                 idx_v, rows_v, None, out_v, None, sg, sw, False)


def _neigh_sum_pair(table_f, table_b, idx_f, idx_b, s128=None):
  with_len = s128 is not None
  scratch = [
      pltpu.VMEM((64, 128), jnp.int32),
      pltpu.VMEM((_NBUF, _CHUNK_ROWS, EMB), jnp.float32),
  ]
  if with_len:
    out_type = (jax.ShapeDtypeStruct((N_NODES, EMB), jnp.float32),
                jax.ShapeDtypeStruct((N_NODES, EMB), jnp.float32),
                jax.ShapeDtypeStruct((N_NODES, 16), jnp.float32),
                jax.ShapeDtypeStruct((N_NODES, 16), jnp.float32))
    scratch.append(pltpu.VMEM((_NBUF, _CHUNK_ROWS, EMB), jnp.float32))
    args = [table_f, idx_f, idx_b, s128]
  else:
    out_type = (jax.ShapeDtypeStruct((N_NODES, EMB), jnp.float32),
                jax.ShapeDtypeStruct((N_NODES, EMB), jnp.float32))
    args = [table_f, table_b, idx_f, idx_b]
  scratch.append(pltpu.VMEM((_NBUF, _NODES_PER_CHUNK, EMB), jnp.float32))
  if with_len:
    scratch.append(pltpu.VMEM((ROWS_PER_W, 16), jnp.float32))
  scratch.append(pltpu.SemaphoreType.DMA((_NBUF,)))
  scratch.append(pltpu.SemaphoreType.DMA((_NBUF,)))
  f = pl.kernel(
      functools.partial(_neigh_sum_body, with_len),
      out_type=out_type,
      mesh=_sc_mesh(),
      scratch_types=scratch,
  )
  return f(*args)


_TBLK = 64
_NGRID = SEQ // _TBLK


def _lstm_body(xsf_ref, xsb_ref, wfT_ref, wbT_ref, bf_ref, bb_ref, wblk_ref,
               ysf_ref, ysb_ref, h_s, c_s, gf_s, gb_s):
  i = pl.program_id(0)

  @pl.when(i == 0)
  def _():
    h_s[...] = jnp.zeros((BATCH, 2 * H_DIR), jnp.float32)
    c_s[...] = jnp.zeros((BATCH, 2 * H_DIR), jnp.float32)

  xf = xsf_ref[...].reshape(_TBLK * BATCH, EMB)
  gf_s[...] = (jnp.dot(xf, wfT_ref[...], preferred_element_type=jnp.float32)
               + bf_ref[0:1, :]).reshape(_TBLK, BATCH, 4 * H_DIR)
  xb = xsb_ref[...].reshape(_TBLK * BATCH, EMB)
  gb_s[...] = (jnp.dot(xb, wbT_ref[...], preferred_element_type=jnp.float32)
               + bb_ref[0:1, :]).reshape(_TBLK, BATCH, 4 * H_DIR)

  def step(k, hc):
    h, c = hc
    tb = _TBLK - 1 - k
    g2 = jnp.dot(h, wblk_ref[...], preferred_element_type=jnp.float32,
                 precision=lax.Precision.DEFAULT)
    gfk = gf_s[k] + g2[:, :4 * H_DIR]
    gbk = gb_s[tb] + g2[:, 4 * H_DIR:]
    hs = []
    cs = []
    for g, c_prev, ys_r, t in ((gfk, c[:, :H_DIR], ysf_ref, k),
                               (gbk, c[:, H_DIR:], ysb_ref, tb)):
      ig = jax.nn.sigmoid(g[:, :H_DIR])
      fg = jax.nn.sigmoid(g[:, H_DIR:2 * H_DIR])
      gg = jnp.tanh(g[:, 2 * H_DIR:3 * H_DIR])
      og = jax.nn.sigmoid(g[:, 3 * H_DIR:])
      c2 = fg * c_prev + ig * gg
      h2 = og * jnp.tanh(c2)
      hs.append(h2)
      cs.append(c2)
      ys_r[t] = h2
    return (jnp.concatenate(hs, axis=1), jnp.concatenate(cs, axis=1))

  hc = lax.fori_loop(0, _TBLK, step, (h_s[...], c_s[...]), unroll=4)
  h_s[...] = hc[0]
  c_s[...] = hc[1]


def _lstm_layer(xs, W_ih, W_hh, b_ih, b_hh, layer):
  wfT = W_ih[layer, 0].T
  wbT = W_ih[layer, 1].T
  bf = jnp.tile((b_ih[layer, 0] + b_hh[layer, 0])[None, :], (8, 1))
  bb = jnp.tile((b_ih[layer, 1] + b_hh[layer, 1])[None, :], (8, 1))
  wblk = jnp.zeros((2 * H_DIR, 8 * H_DIR), jnp.float32)
  wblk = wblk.at[:H_DIR, :4 * H_DIR].set(W_hh[layer, 0].T)
  wblk = wblk.at[H_DIR:, 4 * H_DIR:].set(W_hh[layer, 1].T)

  grid = (_NGRID,)
  blk = pl.BlockSpec((_TBLK, BATCH, EMB), lambda i: (i, 0, 0))
  blk_rev = pl.BlockSpec((_TBLK, BATCH, EMB), lambda i: (_NGRID - 1 - i, 0, 0))
  full = lambda shape: pl.BlockSpec(shape, lambda i: tuple(0 for _ in shape))
  oblk = pl.BlockSpec((_TBLK, BATCH, H_DIR), lambda i: (i, 0, 0))
  oblk_rev = pl.BlockSpec((_TBLK, BATCH, H_DIR),
                          lambda i: (_NGRID - 1 - i, 0, 0))
  return pl.pallas_call(
      _lstm_body,
      grid=grid,
      in_specs=[blk, blk_rev, full((EMB, 4 * H_DIR)), full((EMB, 4 * H_DIR)),
                full((8, 4 * H_DIR)), full((8, 4 * H_DIR)),
                full((2 * H_DIR, 8 * H_DIR))],
      out_specs=[oblk, oblk_rev],
      out_shape=[jax.ShapeDtypeStruct((SEQ, BATCH, H_DIR), jnp.float32),
                 jax.ShapeDtypeStruct((SEQ, BATCH, H_DIR), jnp.float32)],
      scratch_shapes=[pltpu.VMEM((BATCH, 2 * H_DIR), jnp.float32)] * 2
      + [pltpu.VMEM((_TBLK, BATCH, 4 * H_DIR), jnp.float32)] * 2,
      compiler_params=pltpu.CompilerParams(
          dimension_semantics=("arbitrary",)),
  )(xs, xs, wfT, wbT, bf, bb, wblk)


def _row_sign_body(x_ref, ones_ref, out_ref):
  r = jnp.dot(jax.nn.relu(x_ref[...]), ones_ref[...],
              preferred_element_type=jnp.float32)
  out_ref[...] = (r > 0.0).astype(jnp.float32)


def _row_sign(x):
  ones = jnp.ones((EMB, 8), jnp.float32)
  return pl.pallas_call(
      _row_sign_body,
      grid=(32,),
      in_specs=[pl.BlockSpec((512, EMB), lambda i: (i, 0)),
                pl.BlockSpec((EMB, 8), lambda i: (0, 0))],
      out_specs=pl.BlockSpec((512, 8), lambda i: (i, 0)),
      out_shape=jax.ShapeDtypeStruct((N_NODES, 8), jnp.float32),
  )(x, ones)


def _agg_body(h_ref, sums_ref, len_ref, w_ref, b_ref, out_ref):
  recip = 1.0 / jnp.maximum(len_ref[:, 0:1], 1.0)
  means = sums_ref[...] * recip
  g = (jnp.dot(h_ref[...], w_ref[:HIDDEN, :],
               preferred_element_type=jnp.float32)
       + jnp.dot(means, w_ref[HIDDEN:, :], preferred_element_type=jnp.float32)
       + b_ref[0:1, :])
  out_ref[...] = jnp.maximum(g, 0.0)


def _agg(h, sums, len16, W, b):
  b8 = jnp.tile(b[None, :], (8, 1))
  return pl.pallas_call(
      _agg_body,
      grid=(32,),
      in_specs=[pl.BlockSpec((512, HIDDEN), lambda i: (i, 0)),
                pl.BlockSpec((512, HIDDEN), lambda i: (i, 0)),
                pl.BlockSpec((512, 16), lambda i: (i, 0)),
                pl.BlockSpec((2 * HIDDEN, HIDDEN), lambda i: (0, 0)),
                pl.BlockSpec((8, HIDDEN), lambda i: (0, 0))],
      out_specs=pl.BlockSpec((512, HIDDEN), lambda i: (i, 0)),
      out_shape=jax.ShapeDtypeStruct((N_NODES, HIDDEN), jnp.float32),
  )(h, sums, len16, W, b8)


def kernel(fw_adj_info, bw_adj_info, feature_info, batch_nodes, embedding,
           lstm_W_ih, lstm_W_hh, lstm_b_ih, lstm_b_hh, padding_vector,
           fw_agg_W, fw_agg_b, bw_agg_W, bw_agg_b):
  bsz, seq = batch_nodes.shape

  feat2d = feature_info[:-1].reshape(-1).astype(jnp.int32).reshape(128, 128)
  x = _emb_gather(embedding, feat2d)

  xs = x.reshape(bsz, seq, EMB).transpose(1, 0, 2)
  for layer in range(2):
    ysf, ysb = _lstm_layer(xs, lstm_W_ih, lstm_W_hh, lstm_b_ih, lstm_b_hh,
                           layer)
    xs = jnp.concatenate([ysf, ysb], axis=-1)
  feature_vector = xs.transpose(1, 0, 2).reshape(N_NODES, HIDDEN)

  node_repres = jnp.concatenate([feature_vector, padding_vector], axis=0)

  s8 = _row_sign(feature_vector)
  s_pad = (jnp.sum(jax.nn.relu(padding_vector)) > 0.0).astype(jnp.float32)
  s_col = jnp.concatenate([s8[:, :1], s_pad.reshape(1, 1)], axis=0)
  s128 = jnp.pad(s_col, ((0, 0), (0, EMB - 1)))

  fw_idx = fw_adj_info.astype(jnp.int32).reshape(NW * 64, 128)
  bw_idx = bw_adj_info.astype(jnp.int32).reshape(NW * 64, 128)

  sums_f, sums_b, len_f, len_b = _neigh_sum_pair(
      node_repres, None, fw_idx, bw_idx, s128)

  fw_h = _agg(feature_vector, sums_f, len_f, fw_agg_W[0], fw_agg_b[0])
  bw_h = _agg(feature_vector, sums_b, len_b, bw_agg_W[0], bw_agg_b[0])

  zero_row = jnp.zeros((1, HIDDEN), jnp.float32)
  for layer in range(1, N_LAYERS):
    tf = jnp.concatenate([fw_h, zero_row], axis=0)
    tb = jnp.concatenate([bw_h, zero_row], axis=0)
    sums_f, sums_b = _neigh_sum_pair(tf, tb, fw_idx, bw_idx)
    fw_h = _agg(fw_h, sums_f, len_f, fw_agg_W[layer], fw_agg_b[layer])
    bw_h = _agg(bw_h, sums_b, len_b, bw_agg_W[layer], bw_agg_b[layer])

  return jnp.concatenate([fw_h.reshape(bsz, seq, HIDDEN),
                          bw_h.reshape(bsz, seq, HIDDEN)], axis=-1)
